# baseline (speedup 1.0000x reference)
import jax
import jax.numpy as jnp
import numpy as np
from jax.experimental import pallas as pl
from jax.experimental.pallas import tpu as pltpu

_G = 2
_R = 2
_CDIM = 4
_DPG = 512
_HALF_L = 4.0 * (1.0 + 1e-3) / 2.0
_HALF_W = 2.0
_BASIS = (1.0, 5.0, 25.0, 125.0)
_TT = 2048
_CT = 512
_LANES = 128


def _bf16_rtne32(v):
    u = jax.lax.bitcast_convert_type(v, jnp.int32)
    r = jax.lax.shift_right_logical(u, 16) & 1
    u = u + (0x7FFF + r)
    u = u & jnp.int32(-65536)
    return jax.lax.bitcast_convert_type(u, jnp.float32)


def _fsq_rows(z, basis4):
    r0 = jnp.round(jnp.tanh(z) * _HALF_L)
    resid = z - r0 * (1.0 / _HALF_W)
    r1 = jnp.round(jnp.tanh(resid * 4.0) * _HALF_L)
    i0 = jnp.sum((r0 + _HALF_W) * basis4, axis=0, keepdims=True)
    i1 = jnp.sum((r1 + _HALF_W) * basis4, axis=0, keepdims=True)
    return i0, i1


def _fsq_kernel(w0_ref, wrep_ref, b_ref, basis_ref, x_ref, o_ref):
    basis4 = basis_ref[0:4]
    x0 = x_ref[0, 0:_DPG, :]
    z0 = jax.lax.dot_general(
        w0_ref[...].astype(jnp.bfloat16), x0.astype(jnp.bfloat16),
        (((1,), (0,)), ((), ())),
        preferred_element_type=jnp.float32,
    ) + b_ref[0:4]
    i00, i01 = _fsq_rows(z0, basis4)
    o_ref[0, 0:1, :] = i00.astype(jnp.int32)
    o_ref[0, 1:2, :] = i01.astype(jnp.int32)
    for kk in range(_TT // _CT):
        cs = slice(kk * _CT, (kk + 1) * _CT)
        accs = [None] * _CDIM
        for j in range(_DPG // 8):
            xs = _bf16_rtne32(x_ref[0, _DPG + 8 * j:_DPG + 8 * (j + 1), cs])
            for c in range(_CDIM):
                wv = wrep_ref[c, 8 * j:8 * (j + 1), :]
                wt = jnp.tile(wv, (1, _CT // _LANES))
                p = wt * xs
                accs[c] = p if accs[c] is None else accs[c] + p
        for c in range(_CDIM):
            z = jnp.sum(accs[c], axis=0, keepdims=True) + b_ref[4 + c, 0]
            r0 = jnp.round(jnp.tanh(z) * _HALF_L)
            resid = z - r0 * (1.0 / _HALF_W)
            r1 = jnp.round(jnp.tanh(resid * 4.0) * _HALF_L)
            i0 = (r0 + _HALF_W) * _BASIS[c]
            i1 = (r1 + _HALF_W) * _BASIS[c]
            if c == 0:
                idx0, idx1 = i0, i1
            else:
                idx0, idx1 = idx0 + i0, idx1 + i1
        o_ref[0, 2, cs] = idx0[0].astype(jnp.int32)
        o_ref[0, 3, cs] = idx1[0].astype(jnp.int32)


def kernel(x, Win, b_in, Wout, b_out):
    del Wout, b_out
    B, D, T = x.shape
    w0 = Win[0]
    w1 = _bf16_rtne32(Win[1])
    wrep = jnp.broadcast_to(w1[:, :, None], (_CDIM, _DPG, _LANES))
    b8 = jnp.concatenate([b_in[0], b_in[1]]).reshape(_G * _CDIM, 1)
    basis8 = jnp.asarray(_BASIS * _G, dtype=jnp.float32).reshape(_G * _CDIM, 1)
    grid = (B, T // _TT)
    out = pl.pallas_call(
        _fsq_kernel,
        grid=grid,
        in_specs=[
            pl.BlockSpec((_CDIM, _DPG), lambda bi, ti: (0, 0)),
            pl.BlockSpec((_CDIM, _DPG, _LANES), lambda bi, ti: (0, 0, 0)),
            pl.BlockSpec((_G * _CDIM, 1), lambda bi, ti: (0, 0)),
            pl.BlockSpec((_G * _CDIM, 1), lambda bi, ti: (0, 0)),
            pl.BlockSpec((1, D, _TT), lambda bi, ti: (bi, 0, ti)),
        ],
        out_specs=pl.BlockSpec((1, _G * _R, _TT), lambda bi, ti: (bi, 0, ti)),
        out_shape=jax.ShapeDtypeStruct((B, _G * _R, T), jnp.int32),
        compiler_params=pltpu.CompilerParams(
            dimension_semantics=("parallel", "parallel"),
        ),
    )(w0, wrep, b8, basis8, x)
    return out


# speedup vs baseline: 1.2675x; 1.2675x over previous
import jax
import jax.numpy as jnp
import numpy as np
from jax.experimental import pallas as pl
from jax.experimental.pallas import tpu as pltpu

_G = 2
_R = 2
_CDIM = 4
_DPG = 512
_HALF_L = 4.0 * (1.0 + 1e-3) / 2.0
_HALF_W = 2.0
_BASIS = (1.0, 5.0, 25.0, 125.0)
_TT = 2048


def _fsq_kernel(w_ref, b_ref, basis_ref, x_ref, o_ref):
    xb = x_ref[0]
    w = w_ref[...]
    b = b_ref[...]
    z = jax.lax.dot_general(
        w.astype(jnp.bfloat16), xb.astype(jnp.bfloat16), (((1,), (0,)), ((), ())),
        preferred_element_type=jnp.float32,
    ) + b
    r0 = jnp.round(jnp.tanh(z) * _HALF_L)
    resid = z - r0 * (1.0 / _HALF_W)
    r1 = jnp.round(jnp.tanh(resid * 4.0) * _HALF_L)
    basis8 = basis_ref[...]
    w0 = (r0 + _HALF_W) * basis8
    w1 = (r1 + _HALF_W) * basis8
    row = [
        jnp.sum(w0[0:4], axis=0, keepdims=True),
        jnp.sum(w1[0:4], axis=0, keepdims=True),
        jnp.sum(w0[4:8], axis=0, keepdims=True),
        jnp.sum(w1[4:8], axis=0, keepdims=True),
    ]
    o_ref[0] = jnp.concatenate(row, axis=0).astype(jnp.int32)


def kernel(x, Win, b_in, Wout, b_out):
    del Wout, b_out
    B, D, T = x.shape
    w8 = jnp.zeros((_G * _CDIM, D), dtype=jnp.float32)
    w8 = w8.at[0:4, 0:512].set(Win[0]).at[4:8, 512:1024].set(Win[1])
    b8 = jnp.concatenate([b_in[0], b_in[1]]).reshape(_G * _CDIM, 1)
    basis8 = jnp.asarray(_BASIS * _G, dtype=jnp.float32).reshape(_G * _CDIM, 1)
    grid = (B, T // _TT)
    out = pl.pallas_call(
        _fsq_kernel,
        grid=grid,
        in_specs=[
            pl.BlockSpec((_G * _CDIM, D), lambda bi, ti: (0, 0)),
            pl.BlockSpec((_G * _CDIM, 1), lambda bi, ti: (0, 0)),
            pl.BlockSpec((_G * _CDIM, 1), lambda bi, ti: (0, 0)),
            pl.BlockSpec((1, D, _TT), lambda bi, ti: (bi, 0, ti)),
        ],
        out_specs=pl.BlockSpec((1, _G * _R, _TT), lambda bi, ti: (bi, 0, ti)),
        out_shape=jax.ShapeDtypeStruct((B, _G * _R, T), jnp.int32),
        compiler_params=pltpu.CompilerParams(
            dimension_semantics=("parallel", "parallel"),
        ),
    )(w8, b8, basis8, x)
    return out
